# baseline (speedup 1.0000x reference)
import functools

import jax
import jax.numpy as jnp
from jax import lax
from jax.experimental import pallas as pl
from jax.experimental.pallas import tpu as pltpu
from jax.experimental.pallas import tpu_sc as plsc

B, D, H, W = 8, 128, 128, 128
HW = H * W
MI = 23
L = 16
DG = 32
NDG = D // DG
HWB = 1024
NBLK = HW // HWB
NCH = HWB // L
NACC = DG * MI * L


def _sc_body(enc_hbm, msk_hbm, out_hbm, enc_v, msk_v, acc_v):
    wid = lax.axis_index("s") * 2 + lax.axis_index("c")
    b = wid // NDG
    dg = wid % NDG
    d0 = dg * DG

    neg = jnp.full((L,), -jnp.inf, dtype=jnp.float32)

    def init_step(k, _):
        acc_v[pl.ds(k * L, L)] = neg
        return 0

    lax.fori_loop(0, DG * MI, init_step, 0)

    def blk_step(blk, _):
        pltpu.sync_copy(
            enc_hbm.at[b, pl.ds(d0, DG), pl.ds(blk * HWB, HWB)], enc_v)
        pltpu.sync_copy(
            msk_hbm.at[b, :, pl.ds(blk * HWB, HWB)], msk_v)

        def d_step(d, _):
            accs = tuple(
                acc_v[pl.ds((d * MI + i) * L, L)] for i in range(MI))

            def ch_step(c, accs):
                e = enc_v[d, pl.ds(c * L, L)]
                new = []
                for i in range(MI):
                    m = msk_v[i, pl.ds(c * L, L)] > 0
                    new.append(jnp.maximum(accs[i], jnp.where(m, e, neg)))
                return tuple(new)

            accs = lax.fori_loop(0, NCH, ch_step, accs)
            for i in range(MI):
                acc_v[pl.ds((d * MI + i) * L, L)] = accs[i]
            return 0

        lax.fori_loop(0, DG, d_step, 0)
        return 0

    lax.fori_loop(0, NBLK, blk_step, 0)
    pltpu.sync_copy(acc_v, out_hbm.at[b, dg])


def _fold_body(p_ref, out_ref):
    out_ref[...] = jnp.max(p_ref[...], axis=-1)


@jax.jit
def kernel(encoded, masks):
    enc = encoded.reshape(B, D, HW)
    msk = masks[:, 1:, :, :].reshape(B, MI, HW)
    mesh = plsc.VectorSubcoreMesh(core_axis_name="c", subcore_axis_name="s")
    partial = pl.kernel(
        _sc_body,
        out_type=jax.ShapeDtypeStruct((B, NDG, NACC), jnp.float32),
        mesh=mesh,
        scratch_types=[
            pltpu.VMEM((DG, HWB), jnp.float32),
            pltpu.VMEM((MI, HWB), jnp.int32),
            pltpu.VMEM((NACC,), jnp.float32),
        ],
    )(enc, msk)
    folded = pl.pallas_call(
        _fold_body,
        grid=(B * NDG // 8,),
        in_specs=[pl.BlockSpec((8, DG * MI, L), lambda n: (n, 0, 0))],
        out_specs=pl.BlockSpec((8, DG * MI), lambda n: (n, 0)),
        out_shape=jax.ShapeDtypeStruct((B * NDG, DG * MI), jnp.float32),
    )(partial.reshape(B * NDG, DG * MI, L))
    out = folded.reshape(B, NDG, DG, MI).reshape(B, D, MI)
    return out[:, :, :, None]


# speedup vs baseline: 1.9866x; 1.9866x over previous
import functools

import jax
import jax.numpy as jnp
from jax import lax
from jax.experimental import pallas as pl
from jax.experimental.pallas import tpu as pltpu
from jax.experimental.pallas import tpu_sc as plsc

B, D, H, W = 8, 128, 128, 128
HW = H * W
MI = 23

BTC = 6
BSC = B - BTC

WB = 1024
NJ = HW // WB
WBP = 4096
NJP = HW // WBP

L = 16
DG = 8
NDG = D // DG
HWB = 1024
NBLK = HW // HWB
NCH = HWB // L
NACC = DG * MI * L


def _bias_body(msk_ref, bias_ref):
    m = msk_ref[0]
    bias = jnp.where(m > 0, jnp.float32(0), jnp.float32(-jnp.inf))
    bias_ref[0] = bias.astype(jnp.bfloat16)


def _tc_body(enc_ref, bias_ref, out_ref, acc_ref):
    j = pl.program_id(1)

    @pl.when(j == 0)
    def _init():
        acc_ref[...] = jnp.full_like(acc_ref, -jnp.inf)

    enc = enc_ref[0].astype(jnp.bfloat16)
    for i in range(MI):
        bi = jnp.broadcast_to(bias_ref[0, i][None, :], (D, WB))
        masked = enc + bi
        f = jnp.maximum(masked[:, :WB // 2], masked[:, WB // 2:])
        f = jnp.maximum(f[:, :WB // 4], f[:, WB // 4:])
        f = jnp.maximum(f[:, :WB // 8], f[:, WB // 8:])
        acc_ref[i] = jnp.maximum(acc_ref[i], f)

    @pl.when(j == NJ - 1)
    def _finish():
        out_ref[0] = jnp.max(acc_ref[...], axis=-1).astype(jnp.float32)


def _sc_body(enc_hbm, msk_hbm, out_hbm, enc_v, msk_v, acc_v):
    wid = lax.axis_index("s") * 2 + lax.axis_index("c")
    b = BTC + wid // NDG
    dg = wid % NDG
    d0 = dg * DG

    neg = jnp.full((L,), -jnp.inf, dtype=jnp.float32)

    def init_step(k, _):
        acc_v[pl.ds(k * L, L)] = neg
        return 0

    lax.fori_loop(0, DG * MI, init_step, 0)

    def blk_step(blk, _):
        pltpu.sync_copy(
            enc_hbm.at[b, pl.ds(d0, DG), pl.ds(blk * HWB, HWB)], enc_v)
        pltpu.sync_copy(
            msk_hbm.at[b, :, pl.ds(blk * HWB, HWB)], msk_v)

        def d_step(d, _):
            accs = tuple(
                acc_v[pl.ds((d * MI + i) * L, L)] for i in range(MI))

            def ch_step(c, accs):
                e = enc_v[d, pl.ds(c * L, L)]
                new = []
                for i in range(MI):
                    m = msk_v[i, pl.ds(c * L, L)] > 0
                    new.append(jnp.maximum(accs[i], jnp.where(m, e, neg)))
                return tuple(new)

            accs = lax.fori_loop(0, NCH, ch_step, accs)
            for i in range(MI):
                acc_v[pl.ds((d * MI + i) * L, L)] = accs[i]
            return 0

        lax.fori_loop(0, DG, d_step, 0)
        return 0

    lax.fori_loop(0, NBLK, blk_step, 0)
    pltpu.sync_copy(acc_v, out_hbm.at[wid])


def _fold_body(p_ref, out_ref):
    out_ref[...] = jnp.max(p_ref[...], axis=-1)


@jax.jit
def kernel(encoded, masks):
    enc = encoded.reshape(B, D, HW)
    msk = masks[:, 1:, :, :].reshape(B, MI, HW)

    mesh = plsc.VectorSubcoreMesh(core_axis_name="c", subcore_axis_name="s")
    partial = pl.kernel(
        _sc_body,
        out_type=jax.ShapeDtypeStruct((BSC * NDG, NACC), jnp.float32),
        mesh=mesh,
        scratch_types=[
            pltpu.VMEM((DG, HWB), jnp.float32),
            pltpu.VMEM((MI, HWB), jnp.int32),
            pltpu.VMEM((NACC,), jnp.float32),
        ],
    )(enc, msk)

    bias = pl.pallas_call(
        _bias_body,
        grid=(BTC, NJP),
        in_specs=[pl.BlockSpec((1, MI, WBP), lambda b, j: (b, 0, j))],
        out_specs=pl.BlockSpec((1, MI, WBP), lambda b, j: (b, 0, j)),
        out_shape=jax.ShapeDtypeStruct((BTC, MI, HW), jnp.bfloat16),
    )(msk[:BTC])
    out_tc = pl.pallas_call(
        _tc_body,
        grid=(BTC, NJ),
        in_specs=[
            pl.BlockSpec((1, D, WB), lambda b, j: (b, 0, j)),
            pl.BlockSpec((1, MI, WB), lambda b, j: (b, 0, j)),
        ],
        out_specs=pl.BlockSpec((1, MI, D), lambda b, j: (b, 0, 0)),
        out_shape=jax.ShapeDtypeStruct((BTC, MI, D), jnp.float32),
        scratch_shapes=[pltpu.VMEM((MI, D, 128), jnp.bfloat16)],
        compiler_params=pltpu.CompilerParams(
            dimension_semantics=("arbitrary", "arbitrary"),
        ),
    )(enc[:BTC], bias)

    folded = pl.pallas_call(
        _fold_body,
        grid=(BSC * NDG // 8,),
        in_specs=[pl.BlockSpec((8, DG * MI, L), lambda n: (n, 0, 0))],
        out_specs=pl.BlockSpec((8, DG * MI), lambda n: (n, 0)),
        out_shape=jax.ShapeDtypeStruct((BSC * NDG, DG * MI), jnp.float32),
    )(partial.reshape(BSC * NDG, DG * MI, L))

    out_sc = folded.reshape(BSC, NDG, DG, MI).reshape(BSC, D, MI)
    out_tc = jnp.transpose(out_tc, (0, 2, 1))
    out = jnp.concatenate([out_tc, out_sc], axis=0)
    return out[:, :, :, None]
